# baseline (speedup 1.0000x reference)
import jax
import jax.numpy as jnp
from jax import lax
from jax.experimental import pallas as pl
from jax.experimental.pallas import tpu as pltpu
from jax.experimental.pallas import tpu_sc as plsc

_BATCH = 64
_N_PATCHES = 1024
_MODEL_DIM = 768
_LANES = 16

_NUM_WORKERS = 32
_P_PER_W = _N_PATCHES // _NUM_WORKERS
_ROWS = 16
_HALVES = _P_PER_W // _ROWS
_NCHUNK = _BATCH * _HALVES
_VECS_PER_ROW = _MODEL_DIM // _LANES
_NBUF = 4


def _sc_body(patches_hbm, pos_hbm, out_hbm, pos_v, in_bufs, out_bufs,
             in_sems, out_sems):
    nc = 2
    wid = lax.axis_index("s") * nc + lax.axis_index("c")
    p0 = wid * _P_PER_W

    pltpu.sync_copy(pos_hbm.at[pl.ds(p0, _P_PER_W)], pos_v)

    def src_slice(c):
        b = c // _HALVES
        off = (c % _HALVES) * _ROWS
        return patches_hbm.at[b, pl.ds(p0 + off, _ROWS)]

    def dst_slice(c):
        b = c // _HALVES
        off = (c % _HALVES) * _ROWS
        return out_hbm.at[b, pl.ds(p0 + off, _ROWS)]

    def start_in(c, k):
        pltpu.async_copy(src_slice(c), in_bufs[k], in_sems[k])

    def wait_in(c, k):
        pltpu.make_async_copy(src_slice(c), in_bufs[k], in_sems[k]).wait()

    def start_out(c, k):
        pltpu.async_copy(out_bufs[k], dst_slice(c), out_sems[k])

    def wait_out(c, k):
        pltpu.make_async_copy(out_bufs[k], dst_slice(c), out_sems[k]).wait()

    def compute(c, k):
        off = (c % _HALVES) * _ROWS

        def row_step(r, carry):
            for j in range(_VECS_PER_ROW):
                sl = pl.ds(j * _LANES, _LANES)
                out_bufs[k][r, sl] = in_bufs[k][r, sl] + pos_v[off + r, sl]
            return carry
        lax.fori_loop(0, _ROWS, row_step, 0, unroll=False)

    for k in range(_NBUF):
        start_in(k, k)

    def chunk_group(g, carry):
        for k in range(_NBUF):
            c = g + k
            wait_in(c, k)

            @pl.when(g > 0)
            def _():
                wait_out(c - _NBUF, k)

            compute(c, k)
            start_out(c, k)

            @pl.when(c + _NBUF < _NCHUNK)
            def _():
                start_in(c + _NBUF, k)
        return carry

    lax.fori_loop(0, _NCHUNK // _NBUF,
                  lambda i, c: chunk_group(i * _NBUF, c), 0, unroll=False)

    for k in range(_NBUF):
        wait_out(_NCHUNK - _NBUF + k, k)


@jax.jit
def kernel(patches, pos_table):
    mesh = plsc.VectorSubcoreMesh(core_axis_name="c", subcore_axis_name="s")
    return pl.kernel(
        _sc_body,
        out_type=jax.ShapeDtypeStruct((_BATCH, _N_PATCHES, _MODEL_DIM),
                                      jnp.float32),
        mesh=mesh,
        scratch_types=[
            pltpu.VMEM((_P_PER_W, _MODEL_DIM), jnp.float32),
            [pltpu.VMEM((_ROWS, _MODEL_DIM), jnp.float32)
             for _ in range(_NBUF)],
            [pltpu.VMEM((_ROWS, _MODEL_DIM), jnp.float32)
             for _ in range(_NBUF)],
            [pltpu.SemaphoreType.DMA for _ in range(_NBUF)],
            [pltpu.SemaphoreType.DMA for _ in range(_NBUF)],
        ],
        name="pos_embed_add_sc",
    )(patches, pos_table)


# speedup vs baseline: 1.1823x; 1.1823x over previous
import jax
import jax.numpy as jnp
from jax import lax
from jax.experimental import pallas as pl
from jax.experimental.pallas import tpu as pltpu
from jax.experimental.pallas import tpu_sc as plsc

_BATCH = 64
_N_PATCHES = 1024
_MODEL_DIM = 768
_LANES = 16

_NUM_WORKERS = 32
_P_PER_W = _N_PATCHES // _NUM_WORKERS
_BROWS = 32
_BHALVES = _BATCH // _BROWS
_NCHUNK = _P_PER_W * _BHALVES
_VECS_PER_ROW = _MODEL_DIM // _LANES
_NBUF = 2


def _sc_body(patches_hbm, pos_hbm, out_hbm, pos_v, in_bufs, out_bufs,
             in_sems, out_sems):
    nc = 2
    wid = lax.axis_index("s") * nc + lax.axis_index("c")
    p0 = wid * _P_PER_W

    pltpu.sync_copy(pos_hbm.at[pl.ds(p0, _P_PER_W)], pos_v)

    def src_slice(c):
        p = p0 + c // _BHALVES
        b0 = (c % _BHALVES) * _BROWS
        return patches_hbm.at[pl.ds(b0, _BROWS), p]

    def dst_slice(c):
        p = p0 + c // _BHALVES
        b0 = (c % _BHALVES) * _BROWS
        return out_hbm.at[pl.ds(b0, _BROWS), p]

    def start_in(c, k):
        pltpu.async_copy(src_slice(c), in_bufs[k], in_sems[k])

    def wait_in(c, k):
        pltpu.make_async_copy(src_slice(c), in_bufs[k], in_sems[k]).wait()

    def start_out(c, k):
        pltpu.async_copy(out_bufs[k], dst_slice(c), out_sems[k])

    def wait_out(c, k):
        pltpu.make_async_copy(out_bufs[k], dst_slice(c), out_sems[k]).wait()

    def compute(c, k):
        p_local = c // _BHALVES
        pos_vecs = tuple(pos_v[p_local, pl.ds(j * _LANES, _LANES)]
                         for j in range(_VECS_PER_ROW))

        def b_step(i, carry):
            for j in range(_VECS_PER_ROW):
                sl = pl.ds(j * _LANES, _LANES)
                out_bufs[k][i, sl] = in_bufs[k][i, sl] + carry[j]
            return carry
        lax.fori_loop(0, _BROWS, b_step, pos_vecs, unroll=False)

    for k in range(_NBUF):
        start_in(k, k)

    def chunk_group(g, carry):
        for k in range(_NBUF):
            c = g + k
            wait_in(c, k)

            @pl.when(g > 0)
            def _():
                wait_out(c - _NBUF, k)

            compute(c, k)
            start_out(c, k)

            @pl.when(c + _NBUF < _NCHUNK)
            def _():
                start_in(c + _NBUF, k)
        return carry

    lax.fori_loop(0, _NCHUNK // _NBUF,
                  lambda i, c: chunk_group(i * _NBUF, c), 0, unroll=False)

    for k in range(_NBUF):
        wait_out(_NCHUNK - _NBUF + k, k)


@jax.jit
def kernel(patches, pos_table):
    mesh = plsc.VectorSubcoreMesh(core_axis_name="c", subcore_axis_name="s")
    return pl.kernel(
        _sc_body,
        out_type=jax.ShapeDtypeStruct((_BATCH, _N_PATCHES, _MODEL_DIM),
                                      jnp.float32),
        mesh=mesh,
        scratch_types=[
            pltpu.VMEM((_P_PER_W, _MODEL_DIM), jnp.float32),
            [pltpu.VMEM((_BROWS, _MODEL_DIM), jnp.float32)
             for _ in range(_NBUF)],
            [pltpu.VMEM((_BROWS, _MODEL_DIM), jnp.float32)
             for _ in range(_NBUF)],
            [pltpu.SemaphoreType.DMA for _ in range(_NBUF)],
            [pltpu.SemaphoreType.DMA for _ in range(_NBUF)],
        ],
        name="pos_embed_add_sc",
    )(patches, pos_table)
